# Initial kernel scaffold; baseline (speedup 1.0000x reference)
import jax
import jax.numpy as jnp
from jax.experimental import pallas as pl


def kernel(user_emb, item_emb, edge_index):
    raise NotImplementedError("write your pallas kernel here")


# speedup vs baseline: 6.4307x; 6.4307x over previous
import jax
import jax.numpy as jnp
from jax import lax
from jax.experimental import pallas as pl
from jax.experimental.pallas import tpu as pltpu
from jax.experimental.pallas import tpu_sc as plsc

N_USERS = 25000
N_ITEMS = 25000
N = N_USERS + N_ITEMS
EMB = 64
N_LAYERS = 3
NE = 800000

NC = 2
NS = 16
OWN = N // NC
SUBROWS = 1664
A = NS * SUBROWS
TRASH = A - 8
EB = 128
NBLK = 391
NE_PAD = NS * EB * NBLK
PAD_COL = N + 1000

_mesh = plsc.VectorSubcoreMesh(core_axis_name="c", subcore_axis_name="s")
_sc_params = pltpu.CompilerParams(use_tc_tiling_on_sc=False)


def _deg_body(col_hbm, out_hbm, acc, buf, cidx, sidx):
    c = lax.axis_index("c")
    s = lax.axis_index("s")
    base = c * OWN

    for i in range(0, EB, 16):
        buf[pl.ds(i, 16)] = jnp.zeros((16,), jnp.float32)

    @pl.loop(0, SUBROWS, step=EB)
    def _(r):
        pltpu.sync_copy(buf, acc.at[pl.ds(s * SUBROWS + r, EB)])

    plsc.subcore_barrier()

    for i in range(0, EB, 16):
        buf[pl.ds(i, 16)] = jnp.ones((16,), jnp.float32)

    @pl.loop(0, NBLK)
    def _(j):
        e0 = (s * NBLK + j) * EB
        pltpu.sync_copy(col_hbm.at[pl.ds(e0, EB)], cidx)
        for k in range(0, EB, 16):
            cv = cidx[pl.ds(k, 16)]
            loc = cv - base
            ok = (loc >= 0) & (loc < OWN)
            sidx[pl.ds(k, 16)] = jnp.where(ok, loc, TRASH)
        pltpu.sync_copy(buf, acc.at[sidx], add=True)

    plsc.subcore_barrier()
    pltpu.sync_copy(acc.at[pl.ds(s * SUBROWS, SUBROWS)],
                    out_hbm.at[pl.ds(c * A + s * SUBROWS, SUBROWS)])


_deg_call = pl.kernel(
    _deg_body,
    out_type=jax.ShapeDtypeStruct((NC * A,), jnp.float32),
    mesh=_mesh,
    scratch_types=[
        pltpu.VMEM_SHARED((A,), jnp.float32),
        pltpu.VMEM((EB,), jnp.float32),
        pltpu.VMEM((EB,), jnp.int32),
        pltpu.VMEM((EB,), jnp.int32),
    ],
    compiler_params=_sc_params,
)


def _layer_body(xs_hbm, row_hbm, col_hbm, out_hbm, acc, rows, ridx, cidx, sidx):
    c = lax.axis_index("c")
    s = lax.axis_index("s")
    base = c * OWN

    for r in range(EB):
        for k in range(0, EMB, 16):
            rows[r, pl.ds(k, 16)] = jnp.zeros((16,), jnp.float32)

    @pl.loop(0, SUBROWS, step=EB)
    def _(r):
        pltpu.sync_copy(rows, acc.at[pl.ds(s * SUBROWS + r, EB)])

    plsc.subcore_barrier()

    @pl.loop(0, NBLK)
    def _(j):
        e0 = (s * NBLK + j) * EB
        pltpu.sync_copy(row_hbm.at[pl.ds(e0, EB)], ridx)
        pltpu.sync_copy(col_hbm.at[pl.ds(e0, EB)], cidx)
        pltpu.sync_copy(xs_hbm.at[ridx], rows)
        for k in range(0, EB, 16):
            cv = cidx[pl.ds(k, 16)]
            loc = cv - base
            ok = (loc >= 0) & (loc < OWN)
            sidx[pl.ds(k, 16)] = jnp.where(ok, loc, TRASH)
        pltpu.sync_copy(rows, acc.at[sidx], add=True)

    plsc.subcore_barrier()
    pltpu.sync_copy(acc.at[pl.ds(s * SUBROWS, SUBROWS)],
                    out_hbm.at[pl.ds(c * A + s * SUBROWS, SUBROWS)])


_layer_call = pl.kernel(
    _layer_body,
    out_type=jax.ShapeDtypeStruct((NC * A, EMB), jnp.float32),
    mesh=_mesh,
    scratch_types=[
        pltpu.VMEM_SHARED((A, EMB), jnp.float32),
        pltpu.VMEM((EB, EMB), jnp.float32),
        pltpu.VMEM((EB,), jnp.int32),
        pltpu.VMEM((EB,), jnp.int32),
        pltpu.VMEM((EB,), jnp.int32),
    ],
    compiler_params=_sc_params,
)


_R = 2000
_G = N // _R


def _pre_tc(deg_ref, x_ref, dis_ref, xs_ref):
    d = deg_ref[...]
    dis = jnp.where(d > 0.0, lax.rsqrt(d), 0.0)
    dis_ref[...] = dis
    xs_ref[...] = x_ref[...] * dis


def _post_tc(acc_ref, dis_ref, x_ref, xs_ref):
    dis = dis_ref[...]
    x = acc_ref[...] * dis
    x_ref[...] = x
    xs_ref[...] = x * dis


def _fin_tc(acc_ref, dis_ref, x0_ref, x1_ref, x2_ref, x3_ref, mean_ref):
    x3 = acc_ref[...] * dis_ref[...]
    x3_ref[...] = x3
    mean_ref[...] = (x0_ref[...] + x1_ref[...] + x2_ref[...] + x3) * 0.25


def _col_spec(w):
    return pl.BlockSpec((_R, w), lambda i: (i, 0))


_pre_call = pl.pallas_call(
    _pre_tc,
    grid=(_G,),
    in_specs=[_col_spec(1), _col_spec(EMB)],
    out_specs=[_col_spec(1), _col_spec(EMB)],
    out_shape=[jax.ShapeDtypeStruct((N, 1), jnp.float32),
               jax.ShapeDtypeStruct((N, EMB), jnp.float32)],
)

_post_call = pl.pallas_call(
    _post_tc,
    grid=(_G,),
    in_specs=[_col_spec(EMB), _col_spec(1)],
    out_specs=[_col_spec(EMB), _col_spec(EMB)],
    out_shape=[jax.ShapeDtypeStruct((N, EMB), jnp.float32),
               jax.ShapeDtypeStruct((N, EMB), jnp.float32)],
)

_fin_call = pl.pallas_call(
    _fin_tc,
    grid=(_G,),
    in_specs=[_col_spec(EMB), _col_spec(1),
              _col_spec(EMB), _col_spec(EMB), _col_spec(EMB)],
    out_specs=[_col_spec(EMB), _col_spec(EMB)],
    out_shape=[jax.ShapeDtypeStruct((N, EMB), jnp.float32),
               jax.ShapeDtypeStruct((N, EMB), jnp.float32)],
)


def kernel(user_emb, item_emb, edge_index):
    x0 = jnp.concatenate([user_emb, item_emb], axis=0)
    row = edge_index[0].astype(jnp.int32)
    col = edge_index[1].astype(jnp.int32)
    pad = NE_PAD - NE
    rowp = jnp.concatenate([row, jnp.zeros((pad,), jnp.int32)])
    colp = jnp.concatenate([col, jnp.full((pad,), PAD_COL, jnp.int32)])

    degp = _deg_call(colp)
    deg = jnp.concatenate([degp[:OWN], degp[A:A + OWN]])[:, None]

    dis, xs = _pre_call(deg, x0)

    embs = [x0]
    mean = None
    for l in range(N_LAYERS):
        accp = _layer_call(xs, rowp, colp)
        acc = jnp.concatenate([accp[:OWN], accp[A:A + OWN]], axis=0)
        if l < N_LAYERS - 1:
            x, xs = _post_call(acc, dis)
            embs.append(x)
        else:
            x3, mean = _fin_call(acc, dis, embs[0], embs[1], embs[2])
            embs.append(x3)

    all_emb = jnp.stack(embs, axis=1)
    return (mean[:N_USERS], mean[N_ITEMS:], all_emb)


# speedup vs baseline: 16.0064x; 2.4891x over previous
import jax
import jax.numpy as jnp
from jax import lax
from jax.experimental import pallas as pl
from jax.experimental.pallas import tpu as pltpu
from jax.experimental.pallas import tpu_sc as plsc

N_USERS = 25000
N_ITEMS = 25000
N = N_USERS + N_ITEMS
EMB = 64
HALF = EMB // 2
N_LAYERS = 3
NE = 800000

NC = 2
NS = 16
EB = 128
NBLK = 392
NE_PAD = NS * NBLK * EB
NBLK_D = NBLK // NC
SUBROWS = 3200
A = NS * SUBROWS
PAD_COL = A - 8

_mesh = plsc.VectorSubcoreMesh(core_axis_name="c", subcore_axis_name="s")
_sc_params = pltpu.CompilerParams(use_tc_tiling_on_sc=False)


def _deg_body(c_hbm, out_hbm, acc, cidx, buf):
    c = lax.axis_index("c")
    s = lax.axis_index("s")

    for i in range(0, EB, 16):
        buf[pl.ds(i, 16)] = jnp.zeros((16,), jnp.float32)

    @pl.loop(0, SUBROWS, step=EB)
    def _(r):
        pltpu.sync_copy(buf, acc.at[pl.ds(s * SUBROWS + r, EB)])

    plsc.subcore_barrier()

    for i in range(0, EB, 16):
        buf[pl.ds(i, 16)] = jnp.ones((16,), jnp.float32)

    pltpu.sync_copy(c_hbm.at[pl.ds((c * NS + s) * NBLK_D, NBLK_D)], cidx)

    @pl.loop(0, NBLK_D)
    def _(j):
        pltpu.sync_copy(buf, acc.at[cidx.at[j]], add=True)

    plsc.subcore_barrier()
    pltpu.sync_copy(acc.at[pl.ds(s * SUBROWS, SUBROWS)],
                    out_hbm.at[pl.ds(c * A + s * SUBROWS, SUBROWS)])


_deg_call = pl.kernel(
    _deg_body,
    out_type=jax.ShapeDtypeStruct((NC * A,), jnp.float32),
    mesh=_mesh,
    scratch_types=[
        pltpu.VMEM_SHARED((A,), jnp.float32),
        pltpu.VMEM((NBLK_D, EB), jnp.int32),
        pltpu.VMEM((EB,), jnp.float32),
    ],
    compiler_params=_sc_params,
)


K = 28
NCHUNK = NBLK // K


def _layer_body(xs2_hbm, g_hbm, c_hbm, out_hbm, acc, gch, cch, rows,
                sg0, sg1, si0, si1):
    c = lax.axis_index("c")
    s = lax.axis_index("s")
    gbase = (c * NS + s) * NBLK
    cbase = s * NBLK

    for r in range(EB):
        for k in range(0, HALF, 16):
            rows[0, r, pl.ds(k, 16)] = jnp.zeros((16,), jnp.float32)

    @pl.loop(0, SUBROWS, step=EB)
    def _(r):
        pltpu.sync_copy(rows.at[0], acc.at[pl.ds(s * SUBROWS + r, EB)])

    plsc.subcore_barrier()

    pltpu.async_copy(g_hbm.at[pl.ds(gbase, K)], gch.at[0], si0)
    pltpu.async_copy(c_hbm.at[pl.ds(cbase, K)], cch.at[0], si0)
    pltpu.async_copy(g_hbm.at[pl.ds(gbase + K, K)], gch.at[1], si1)
    pltpu.async_copy(c_hbm.at[pl.ds(cbase + K, K)], cch.at[1], si1)

    def chunk(i, slot, si_this):
        pltpu.make_async_copy(g_hbm.at[pl.ds(gbase, K)], gch.at[slot], si_this).wait()
        pltpu.make_async_copy(c_hbm.at[pl.ds(cbase, K)], cch.at[slot], si_this).wait()

        pltpu.async_copy(xs2_hbm.at[gch.at[slot, 0]], rows.at[0], sg0)

        @pl.loop(0, K, step=2)
        def _(k):
            pltpu.async_copy(xs2_hbm.at[gch.at[slot, k + 1]], rows.at[1], sg1)
            pltpu.make_async_copy(xs2_hbm.at[gch.at[slot, k]], rows.at[0], sg0).wait()
            pltpu.sync_copy(rows.at[0], acc.at[cch.at[slot, k]], add=True)

            @pl.when(k + 2 < K)
            def _():
                pltpu.async_copy(xs2_hbm.at[gch.at[slot, k + 2]], rows.at[0], sg0)

            pltpu.make_async_copy(xs2_hbm.at[gch.at[slot, k + 1]], rows.at[1], sg1).wait()
            pltpu.sync_copy(rows.at[1], acc.at[cch.at[slot, k + 1]], add=True)

        @pl.when(i + 2 < NCHUNK)
        def _():
            pltpu.async_copy(g_hbm.at[pl.ds(gbase + (i + 2) * K, K)], gch.at[slot], si_this)
            pltpu.async_copy(c_hbm.at[pl.ds(cbase + (i + 2) * K, K)], cch.at[slot], si_this)

    @pl.loop(0, NCHUNK, step=2)
    def _(i):
        chunk(i, 0, si0)
        chunk(i + 1, 1, si1)

    plsc.subcore_barrier()
    pltpu.sync_copy(acc.at[pl.ds(s * SUBROWS, SUBROWS)],
                    out_hbm.at[pl.ds(c * A + s * SUBROWS, SUBROWS)])


_layer_call = pl.kernel(
    _layer_body,
    out_type=jax.ShapeDtypeStruct((NC * A, HALF), jnp.float32),
    mesh=_mesh,
    scratch_types=[
        pltpu.VMEM_SHARED((A, HALF), jnp.float32),
        pltpu.VMEM((2, K, EB), jnp.int32),
        pltpu.VMEM((2, K, EB), jnp.int32),
        pltpu.VMEM((2, EB, HALF), jnp.float32),
        pltpu.SemaphoreType.DMA,
        pltpu.SemaphoreType.DMA,
        pltpu.SemaphoreType.DMA,
        pltpu.SemaphoreType.DMA,
    ],
    compiler_params=_sc_params,
)


_R = 2000
_G = N // _R


def _pre_tc(d0_ref, d1_ref, x_ref, dis_ref, xs_ref):
    d = d0_ref[...] + d1_ref[...]
    dis = jnp.where(d > 0.0, lax.rsqrt(d), 0.0)
    dis_ref[...] = dis
    xs_ref[...] = x_ref[...] * dis


def _post_tc(a0_ref, a1_ref, dis_ref, x_ref, xs_ref):
    dis = dis_ref[...]
    x = jnp.concatenate([a0_ref[...], a1_ref[...]], axis=1) * dis
    x_ref[...] = x
    xs_ref[...] = x * dis


def _fin_tc(a0_ref, a1_ref, dis_ref, x0_ref, x1_ref, x2_ref, x3_ref, mean_ref):
    x3 = jnp.concatenate([a0_ref[...], a1_ref[...]], axis=1) * dis_ref[...]
    x3_ref[...] = x3
    mean_ref[...] = (x0_ref[...] + x1_ref[...] + x2_ref[...] + x3) * 0.25


def _col_spec(w):
    return pl.BlockSpec((_R, w), lambda i: (i, 0))


_pre_call = pl.pallas_call(
    _pre_tc,
    grid=(_G,),
    in_specs=[_col_spec(1), _col_spec(1), _col_spec(EMB)],
    out_specs=[_col_spec(1), _col_spec(EMB)],
    out_shape=[jax.ShapeDtypeStruct((N, 1), jnp.float32),
               jax.ShapeDtypeStruct((N, EMB), jnp.float32)],
)

_post_call = pl.pallas_call(
    _post_tc,
    grid=(_G,),
    in_specs=[_col_spec(HALF), _col_spec(HALF), _col_spec(1)],
    out_specs=[_col_spec(EMB), _col_spec(EMB)],
    out_shape=[jax.ShapeDtypeStruct((N, EMB), jnp.float32),
               jax.ShapeDtypeStruct((N, EMB), jnp.float32)],
)

_fin_call = pl.pallas_call(
    _fin_tc,
    grid=(_G,),
    in_specs=[_col_spec(HALF), _col_spec(HALF), _col_spec(1),
              _col_spec(EMB), _col_spec(EMB), _col_spec(EMB)],
    out_specs=[_col_spec(EMB), _col_spec(EMB)],
    out_shape=[jax.ShapeDtypeStruct((N, EMB), jnp.float32),
               jax.ShapeDtypeStruct((N, EMB), jnp.float32)],
)


def kernel(user_emb, item_emb, edge_index):
    x0 = jnp.concatenate([user_emb, item_emb], axis=0)
    row = edge_index[0].astype(jnp.int32)
    col = edge_index[1].astype(jnp.int32)
    pad = NE_PAD - NE
    rowp = jnp.concatenate([row, jnp.zeros((pad,), jnp.int32)])
    colp = jnp.concatenate([col, jnp.full((pad,), PAD_COL, jnp.int32)])
    colb = colp.reshape(NS * NBLK, EB)
    gb = ((2 * rowp)[None, :] +
          jnp.arange(NC, dtype=jnp.int32)[:, None]).reshape(NC * NS * NBLK, EB)

    degp = _deg_call(colb)
    d0 = degp[:N, None]
    d1 = degp[A:A + N, None]
    dis, xs = _pre_call(d0, d1, x0)

    embs = [x0]
    mean = None
    for l in range(N_LAYERS):
        outf = _layer_call(xs.reshape(2 * N, HALF), gb, colb)
        a0 = outf[:N]
        a1 = outf[A:A + N]
        if l < N_LAYERS - 1:
            x, xs = _post_call(a0, a1, dis)
            embs.append(x)
        else:
            x3, mean = _fin_call(a0, a1, dis, embs[0], embs[1], embs[2])
            embs.append(x3)

    all_emb = jnp.stack(embs, axis=1)
    return (mean[:N_USERS], mean[N_ITEMS:], all_emb)
